# baseline (speedup 1.0000x reference)
import jax
import jax.numpy as jnp
from jax import lax
from jax.experimental import pallas as pl
from jax.experimental.pallas import tpu as pltpu
from jax.experimental.pallas import tpu_sc as plsc

N_NODES = 10000
D_FEAT = 256
NUM_PATHS = 50000
PATH_LEN = 4

TOTAL_ROWS = NUM_PATHS * PATH_LEN
CHUNK = 80
NUM_CHUNKS = TOTAL_ROWS // CHUNK
NUM_WORKERS = 32
BASE_CHUNKS = NUM_CHUNKS // NUM_WORKERS
EXTRA = NUM_CHUNKS - BASE_CHUNKS * NUM_WORKERS
MAX_CHUNKS_W = BASE_CHUNKS + 1
NBUF = 4
SDELAY = 1
STEPS = (MAX_CHUNKS_W + SDELAY + NBUF - 1) // NBUF
IDX_PAD = ((NUM_WORKERS - 1) * BASE_CHUNKS + EXTRA + MAX_CHUNKS_W) * CHUNK
LANES = 16
COLS32 = D_FEAT // 32


def _gather_body(idx_hbm, x_hbm, out_hbm, idx_v, raw_v, rows_v, gsem, ssem):
    nc = jnp.int32(2)
    wid = lax.axis_index("s") * nc + lax.axis_index("c")
    count = jnp.int32(BASE_CHUNKS) + jnp.where(wid < EXTRA, 1, 0).astype(jnp.int32)
    start = jnp.int32(BASE_CHUNKS) * wid + jnp.minimum(wid, jnp.int32(EXTRA))
    pltpu.sync_copy(
        idx_hbm.at[pl.ds(start * jnp.int32(CHUNK), MAX_CHUNKS_W * CHUNK)], idx_v)

    def gather_args(j, b):
        bi = jnp.int32(b)
        idx_c = idx_v.at[pl.ds(j * jnp.int32(CHUNK), CHUNK)]
        return x_hbm.at[idx_c], raw_v.at[bi], gsem.at[bi]

    def store_args(j, b):
        bi = jnp.int32(b)
        row0 = (start + j) * jnp.int32(CHUNK)
        return rows_v.at[bi], out_hbm.at[pl.ds(row0, CHUNK)], ssem.at[bi]

    def gather(j, b):
        pltpu.async_copy(*gather_args(j, b))

    def gather_wait(j, b):
        pltpu.make_async_copy(*gather_args(j, b)).wait()

    def store(j, b):
        pltpu.async_copy(*store_args(j, b))

    def store_wait(j, b):
        pltpu.make_async_copy(*store_args(j, b)).wait()

    himask = jnp.full((LANES,), -65536, jnp.int32)
    shl16 = jnp.full((LANES,), 16, jnp.int32)

    def widen(b):
        bi = jnp.int32(b)

        @pl.loop(jnp.int32(0), jnp.int32(CHUNK // 8))
        def row8(rb):
            for u in range(8):
                r32 = rb * jnp.int32(8) + jnp.int32(u)
                src = raw_v.at[bi, r32]
                dst = rows_v.at[bi, r32]
                for c in range(COLS32):
                    w = src[pl.ds(c * 16, 16)]
                    dst[pl.ds(c * 32, 16)] = w << shl16
                    dst[pl.ds(c * 32 + 16, 16)] = w & himask

    for b in range(NBUF):
        gather(jnp.int32(b), b)

    @pl.loop(jnp.int32(0), jnp.int32(STEPS))
    def step(s):
        jbase = s * jnp.int32(NBUF)
        for b in range(NBUF):
            j = jbase + jnp.int32(b)

            @pl.when(j < count)
            def _():
                gather_wait(j, b)
                widen(b)
                store(j, b)

            jp = j - jnp.int32(SDELAY)
            bp = (b - SDELAY) % NBUF
            jn = jp + jnp.int32(NBUF)

            @pl.when((jp >= 0) & (jp < count))
            def _():
                store_wait(jp, bp)

            @pl.when((jn >= jnp.int32(NBUF)) & (jn < count))
            def _():
                gather(jn, bp)


@jax.jit
def _sc_gather(idx_flat, x16):
    mesh = plsc.VectorSubcoreMesh(core_axis_name="c", subcore_axis_name="s")
    f = pl.kernel(
        _gather_body,
        mesh=mesh,
        out_type=jax.ShapeDtypeStruct((TOTAL_ROWS, D_FEAT), jnp.int32),
        scratch_types=[
            pltpu.VMEM((MAX_CHUNKS_W * CHUNK,), jnp.int32),
            pltpu.VMEM((NBUF, CHUNK, D_FEAT // 2), jnp.int32),
            pltpu.VMEM((NBUF, CHUNK, D_FEAT), jnp.int32),
            pltpu.SemaphoreType.DMA((NBUF,)),
            pltpu.SemaphoreType.DMA((NBUF,)),
        ],
    )
    return f(idx_flat, x16)


def kernel(path_index_without_target, x, att):
    del att
    idx = path_index_without_target.T.reshape(-1).astype(jnp.int32)
    idx = jnp.pad(idx, (0, IDX_PAD - TOTAL_ROWS))
    x16 = x.astype(jnp.bfloat16).reshape(N_NODES, COLS32, 2, LANES)
    pair = jnp.stack([x16[:, :, 0, :], x16[:, :, 1, :]], axis=-1)
    xp = jax.lax.bitcast_convert_type(pair, jnp.int32).reshape(
        N_NODES, D_FEAT // 2)
    out = _sc_gather(idx, xp)
    out = jax.lax.bitcast_convert_type(out, jnp.float32)
    return out.reshape(PATH_LEN, NUM_PATHS, D_FEAT)


# speedup vs baseline: 1.4262x; 1.4262x over previous
import jax
import jax.numpy as jnp
from jax import lax
from jax.experimental import pallas as pl
from jax.experimental.pallas import tpu as pltpu
from jax.experimental.pallas import tpu_sc as plsc

N_NODES = 10000
D_FEAT = 256
NUM_PATHS = 50000
PATH_LEN = 4

TOTAL_ROWS = NUM_PATHS * PATH_LEN
CHUNK = 80
NUM_CHUNKS = TOTAL_ROWS // CHUNK
NUM_WORKERS = 32
BASE_CHUNKS = NUM_CHUNKS // NUM_WORKERS
EXTRA = NUM_CHUNKS - BASE_CHUNKS * NUM_WORKERS
MAX_CHUNKS_W = BASE_CHUNKS + 1
NBUF = 4
SDELAY = 1
STEPS = (MAX_CHUNKS_W + SDELAY + NBUF - 1) // NBUF
IDX_PAD = ((NUM_WORKERS - 1) * BASE_CHUNKS + EXTRA + MAX_CHUNKS_W) * CHUNK
LANES = 16
COLS32 = D_FEAT // 32


def _gather_body(idx_hbm, x_hbm, out_hbm, idx_v, raw_v, rows_v, gsem, ssem):
    nc = jnp.int32(2)
    wid = lax.axis_index("s") * nc + lax.axis_index("c")
    count = jnp.int32(BASE_CHUNKS) + jnp.where(wid < EXTRA, 1, 0).astype(jnp.int32)
    start = jnp.int32(BASE_CHUNKS) * wid + jnp.minimum(wid, jnp.int32(EXTRA))
    pltpu.sync_copy(
        idx_hbm.at[pl.ds(start * jnp.int32(CHUNK), MAX_CHUNKS_W * CHUNK)], idx_v)

    def gather_args(j, b):
        bi = jnp.int32(b)
        idx_c = idx_v.at[pl.ds(j * jnp.int32(CHUNK), CHUNK)]
        return x_hbm.at[idx_c], raw_v.at[bi], gsem.at[bi]

    def store_args(j, b):
        bi = jnp.int32(b)
        row0 = (start + j) * jnp.int32(CHUNK)
        return rows_v.at[bi], out_hbm.at[pl.ds(row0, CHUNK)], ssem.at[bi]

    def gather(j, b):
        pltpu.async_copy(*gather_args(j, b))

    def gather_wait(j, b):
        pltpu.make_async_copy(*gather_args(j, b)).wait()

    def store(j, b):
        pltpu.async_copy(*store_args(j, b))

    def store_wait(j, b):
        pltpu.make_async_copy(*store_args(j, b)).wait()

    himask = jnp.full((LANES,), -65536, jnp.int32)
    shl16 = jnp.full((LANES,), 16, jnp.int32)

    def widen(b):
        bi = jnp.int32(b)

        @plsc.parallel_loop(jnp.int32(0), jnp.int32(CHUNK), jnp.int32(1), unroll=4)
        def row(r32):
            src = raw_v.at[bi, r32]
            dst = rows_v.at[bi, r32]
            for c in range(COLS32):
                w = src[pl.ds(c * 16, 16)]
                dst[pl.ds(c * 32, 16)] = w << shl16
                dst[pl.ds(c * 32 + 16, 16)] = w & himask

    for b in range(NBUF):
        gather(jnp.int32(b), b)

    @pl.loop(jnp.int32(0), jnp.int32(STEPS))
    def step(s):
        jbase = s * jnp.int32(NBUF)
        for b in range(NBUF):
            j = jbase + jnp.int32(b)

            @pl.when(j < count)
            def _():
                gather_wait(j, b)
                widen(b)
                store(j, b)

            jp = j - jnp.int32(SDELAY)
            bp = (b - SDELAY) % NBUF
            jn = jp + jnp.int32(NBUF)

            @pl.when((jp >= 0) & (jp < count))
            def _():
                store_wait(jp, bp)

            @pl.when((jn >= jnp.int32(NBUF)) & (jn < count))
            def _():
                gather(jn, bp)


@jax.jit
def _sc_gather(idx_flat, x16):
    mesh = plsc.VectorSubcoreMesh(core_axis_name="c", subcore_axis_name="s")
    f = pl.kernel(
        _gather_body,
        mesh=mesh,
        out_type=jax.ShapeDtypeStruct((TOTAL_ROWS, D_FEAT), jnp.int32),
        scratch_types=[
            pltpu.VMEM((MAX_CHUNKS_W * CHUNK,), jnp.int32),
            pltpu.VMEM((NBUF, CHUNK, D_FEAT // 2), jnp.int32),
            pltpu.VMEM((NBUF, CHUNK, D_FEAT), jnp.int32),
            pltpu.SemaphoreType.DMA((NBUF,)),
            pltpu.SemaphoreType.DMA((NBUF,)),
        ],
    )
    return f(idx_flat, x16)


def kernel(path_index_without_target, x, att):
    del att
    idx = path_index_without_target.T.reshape(-1).astype(jnp.int32)
    idx = jnp.pad(idx, (0, IDX_PAD - TOTAL_ROWS))
    x16 = x.astype(jnp.bfloat16).reshape(N_NODES, COLS32, 2, LANES)
    pair = jnp.stack([x16[:, :, 0, :], x16[:, :, 1, :]], axis=-1)
    xp = jax.lax.bitcast_convert_type(pair, jnp.int32).reshape(
        N_NODES, D_FEAT // 2)
    out = _sc_gather(idx, xp)
    out = jax.lax.bitcast_convert_type(out, jnp.float32)
    return out.reshape(PATH_LEN, NUM_PATHS, D_FEAT)


# speedup vs baseline: 2.4070x; 1.6876x over previous
import jax
import jax.numpy as jnp
from jax import lax
from jax.experimental import pallas as pl
from jax.experimental.pallas import tpu as pltpu
from jax.experimental.pallas import tpu_sc as plsc

N_NODES = 10000
D_FEAT = 256
NUM_PATHS = 50000
PATH_LEN = 4

TOTAL_ROWS = NUM_PATHS * PATH_LEN
CHUNK = 80
NUM_CHUNKS = TOTAL_ROWS // CHUNK
NUM_WORKERS = 32
BASE_CHUNKS = NUM_CHUNKS // NUM_WORKERS
EXTRA = NUM_CHUNKS - BASE_CHUNKS * NUM_WORKERS
MAX_CHUNKS_W = BASE_CHUNKS + 1
NBUF = 5
SDELAY = 2
STEPS = (MAX_CHUNKS_W + SDELAY + NBUF - 1) // NBUF
IDX_PAD = ((NUM_WORKERS - 1) * BASE_CHUNKS + EXTRA + MAX_CHUNKS_W) * CHUNK


def _gather_body(idx_hbm, x_hbm, out_hbm, idx_v, rows_v, gsem, ssem):
    nc = jnp.int32(2)
    wid = lax.axis_index("s") * nc + lax.axis_index("c")
    count = jnp.int32(BASE_CHUNKS) + jnp.where(wid < EXTRA, 1, 0).astype(jnp.int32)
    start = jnp.int32(BASE_CHUNKS) * wid + jnp.minimum(wid, jnp.int32(EXTRA))
    pltpu.sync_copy(
        idx_hbm.at[pl.ds(start * jnp.int32(CHUNK), MAX_CHUNKS_W * CHUNK)], idx_v)

    def gather_args(j, b):
        bi = jnp.int32(b)
        idx_c = idx_v.at[pl.ds(j * jnp.int32(CHUNK), CHUNK)]
        return x_hbm.at[idx_c], rows_v.at[bi], gsem.at[bi]

    def store_args(j, b):
        bi = jnp.int32(b)
        row0 = (start + j) * jnp.int32(CHUNK)
        return rows_v.at[bi], out_hbm.at[pl.ds(row0, CHUNK)], ssem.at[bi]

    def gather(j, b):
        pltpu.async_copy(*gather_args(j, b))

    def gather_wait(j, b):
        pltpu.make_async_copy(*gather_args(j, b)).wait()

    def store(j, b):
        pltpu.async_copy(*store_args(j, b))

    def store_wait(j, b):
        pltpu.make_async_copy(*store_args(j, b)).wait()

    for b in range(NBUF):
        gather(jnp.int32(b), b)

    @pl.loop(jnp.int32(0), jnp.int32(STEPS))
    def step(s):
        jbase = s * jnp.int32(NBUF)
        for b in range(NBUF):
            j = jbase + jnp.int32(b)

            @pl.when(j < count)
            def _():
                gather_wait(j, b % NBUF)
                store(j, b % NBUF)

            jp = j - jnp.int32(SDELAY)
            bp = (b - SDELAY) % NBUF
            jn = jp + jnp.int32(NBUF)

            @pl.when((jp >= 0) & (jp < count))
            def _():
                store_wait(jp, bp)

            @pl.when((jn >= jnp.int32(NBUF)) & (jn < count))
            def _():
                gather(jn, bp)


@jax.jit
def _sc_gather(idx_flat, x):
    mesh = plsc.VectorSubcoreMesh(core_axis_name="c", subcore_axis_name="s")
    f = pl.kernel(
        _gather_body,
        mesh=mesh,
        out_type=jax.ShapeDtypeStruct((TOTAL_ROWS, D_FEAT), jnp.float32),
        scratch_types=[
            pltpu.VMEM((MAX_CHUNKS_W * CHUNK,), jnp.int32),
            pltpu.VMEM((NBUF, CHUNK, D_FEAT), jnp.float32),
            pltpu.SemaphoreType.DMA((NBUF,)),
            pltpu.SemaphoreType.DMA((NBUF,)),
        ],
    )
    return f(idx_flat, x)


def kernel(path_index_without_target, x, att):
    del att
    idx = path_index_without_target.T.reshape(-1).astype(jnp.int32)
    idx = jnp.pad(idx, (0, IDX_PAD - TOTAL_ROWS))
    out = _sc_gather(idx, x.astype(jnp.float32))
    return out.reshape(PATH_LEN, NUM_PATHS, D_FEAT)
